# baseline (speedup 1.0000x reference)
import jax
import jax.numpy as jnp
from jax import lax
from jax.experimental import pallas as pl
from jax.experimental.pallas import tpu as pltpu
from jax.experimental.pallas import tpu_sc as plsc

NUM_HEADS = 16
MAX_DIST = 128
S = 2048
LANES = 16
NCLS = 50
CPAD = 576
ROWT = S // 8
COLT = S // 128
NCHUNK = 105
NBAND = 368


def _rpb_body(cols_hbm, out_hbm, col_v, tiles_v, lconst_v, rconst_v, sem):
    c = lax.axis_index("c")
    s = lax.axis_index("s")
    h = s

    pltpu.sync_copy(cols_hbm.at[pl.ds(h * CPAD, CPAD)], col_v)

    def build_body(n, carry):
        cls = n // 64 - 32
        i = (n // 8) % 8
        jj = n % 8
        vals = col_v[pl.ds(288 + 8 * cls - i + jj * LANES, LANES)]
        tiles_v[n // 64, i, pl.ds(jj * LANES, LANES)] = vals
        return carry

    lax.fori_loop(0, NCLS * 64, build_body, 0)

    left = col_v[pl.ds(0, LANES)]
    right = col_v[pl.ds(CPAD - LANES, LANES)]

    def const_body(n, carry):
        i = n // 8
        jj = n % 8
        lconst_v[i, pl.ds(jj * LANES, LANES)] = left
        rconst_v[i, pl.ds(jj * LANES, LANES)] = right
        return carry

    lax.fori_loop(0, 128 * 8, const_body, 0)

    half = ROWT // 2
    a0 = c * half
    a1 = a0 + half

    for b in range(COLT):
        col = pl.multiple_of(b * 128, 128)

        r_hi = jnp.maximum(jnp.minimum(a1, 16 * b - 16), a0)
        n_r = (r_hi - a0) // 16

        def rconst_body(t, carry, col=col):
            row = pl.multiple_of((a0 + 16 * t) * 8, 8)
            pltpu.async_copy(
                rconst_v, out_hbm.at[0, h, pl.ds(row, 128), pl.ds(col, 128)],
                sem)
            return carry

        l_lo = jnp.minimum(jnp.maximum(a0, 16 * b + 32), a1)
        n_l = (a1 - l_lo) // 16

        def lconst_body(t, carry, col=col):
            row = pl.multiple_of((l_lo + 16 * t) * 8, 8)
            pltpu.async_copy(
                lconst_v, out_hbm.at[0, h, pl.ds(row, 128), pl.ds(col, 128)],
                sem)
            return carry

        band_lo = jnp.maximum(a0, 16 * b - 16)
        band_hi = jnp.minimum(a1, 16 * b + 32)

        def band_body(a, carry, b=b, col=col):
            cls_idx = 16 * b - a + 32
            row = pl.multiple_of(a * 8, 8)
            pltpu.async_copy(
                tiles_v.at[cls_idx],
                out_hbm.at[0, h, pl.ds(row, 8), pl.ds(col, 128)],
                sem)
            return carry

        lax.fori_loop(0, n_r, rconst_body, 0)
        lax.fori_loop(0, n_l, lconst_body, 0)
        lax.fori_loop(band_lo, band_hi, band_body, 0)

    def drain_chunk(t, carry):
        pltpu.make_async_copy(
            out_hbm.at[0, 0, pl.ds(0, 128), pl.ds(0, 128)], lconst_v, sem
        ).wait()
        return carry

    def drain_tile(t, carry):
        pltpu.make_async_copy(
            out_hbm.at[0, 0, pl.ds(0, 8), pl.ds(0, 128)], tiles_v.at[0], sem
        ).wait()
        return carry

    lax.fori_loop(0, NCHUNK, drain_chunk, 0)
    lax.fori_loop(0, NBAND, drain_tile, 0)


def kernel(seq_length, table):
    del seq_length
    cols = jnp.pad(table.T, ((0, 0), (160, CPAD - 160 - (2 * MAX_DIST + 1))),
                   mode="edge").reshape(-1)
    mesh = plsc.VectorSubcoreMesh(core_axis_name="c", subcore_axis_name="s")
    f = pl.kernel(
        _rpb_body,
        mesh=mesh,
        out_type=jax.ShapeDtypeStruct((1, NUM_HEADS, S, S), jnp.float32),
        scratch_types=[
            pltpu.VMEM((CPAD,), jnp.float32),
            pltpu.VMEM((NCLS, 8, 128), jnp.float32),
            pltpu.VMEM((128, 128), jnp.float32),
            pltpu.VMEM((128, 128), jnp.float32),
            pltpu.SemaphoreType.DMA,
        ],
        compiler_params=pltpu.CompilerParams(use_tc_tiling_on_sc=True),
    )
    return f(cols)
